# Initial kernel scaffold; baseline (speedup 1.0000x reference)
import jax
import jax.numpy as jnp
from jax.experimental import pallas as pl


def kernel(t, gamma):
    raise NotImplementedError("write your pallas kernel here")


# speedup vs baseline: 4.4858x; 4.4858x over previous
import jax
import jax.numpy as jnp
from jax import lax
from jax.experimental import pallas as pl
from jax.experimental.pallas import tpu as pltpu
from jax.experimental.pallas import tpu_sc as plsc

_N = 16384
_TABLE = 1001
_TABLE_PAD = 1008
_L = 16
_NC = 2
_NS = 16
_NW = _NC * _NS
_CHUNK = _N // _NW
_MAGIC = 8388608.0


def _body(t_hbm, gamma_hbm, out_hbm, t_v, gamma_v, out_v):
    wid = lax.axis_index("s") * _NC + lax.axis_index("c")
    base = wid * _CHUNK
    pltpu.sync_copy(gamma_hbm, gamma_v)
    pltpu.sync_copy(t_hbm.at[pl.ds(base, _CHUNK)], t_v)
    for i in range(_CHUNK // _L):
        sl = pl.ds(i * _L, _L)
        x = t_v[sl] * 1000.0
        r = (x + _MAGIC) - _MAGIC
        idx = r.astype(jnp.int32)
        out_v[sl] = plsc.load_gather(gamma_v, [idx])
    pltpu.sync_copy(out_v, out_hbm.at[pl.ds(base, _CHUNK)])


def kernel(t, gamma):
    gamma_p = jnp.pad(gamma, (0, _TABLE_PAD - _TABLE))
    mesh = plsc.VectorSubcoreMesh(core_axis_name="c", subcore_axis_name="s")
    run = pl.kernel(
        _body,
        mesh=mesh,
        out_type=jax.ShapeDtypeStruct((_N,), jnp.float32),
        scratch_types=[
            pltpu.VMEM((_CHUNK,), jnp.float32),
            pltpu.VMEM((_TABLE_PAD,), jnp.float32),
            pltpu.VMEM((_CHUNK,), jnp.float32),
        ],
        compiler_params=pltpu.CompilerParams(needs_layout_passes=False),
    )
    return run(t, gamma_p)


# speedup vs baseline: 4.6131x; 1.0284x over previous
import jax
import jax.numpy as jnp
from jax import lax
from jax.experimental import pallas as pl
from jax.experimental.pallas import tpu as pltpu
from jax.experimental.pallas import tpu_sc as plsc

_N = 16384
_TABLE = 1001
_TABLE_PAD = 1008
_L = 16
_NC = 2
_NS = 16
_NW = _NC * _NS
_CHUNK = _N // _NW
_MAGIC = 8388608.0


def _body(t_hbm, gamma_hbm, out_hbm, t_v, gamma_v, out_v, sem_g, sem_t):
    wid = lax.axis_index("s") * _NC + lax.axis_index("c")
    base = wid * _CHUNK
    g_cp = pltpu.async_copy(gamma_hbm, gamma_v, sem_g)
    t_cp = pltpu.async_copy(t_hbm.at[pl.ds(base, _CHUNK)], t_v, sem_t)
    t_cp.wait()
    g_cp.wait()
    for i in range(_CHUNK // _L):
        sl = pl.ds(i * _L, _L)
        x = t_v[sl] * 1000.0
        r = (x + _MAGIC) - _MAGIC
        idx = r.astype(jnp.int32)
        out_v[sl] = plsc.load_gather(gamma_v, [idx])
    pltpu.sync_copy(out_v, out_hbm.at[pl.ds(base, _CHUNK)])


def kernel(t, gamma):
    mesh = plsc.VectorSubcoreMesh(core_axis_name="c", subcore_axis_name="s")
    run = pl.kernel(
        _body,
        mesh=mesh,
        out_type=jax.ShapeDtypeStruct((_N,), jnp.float32),
        scratch_types=[
            pltpu.VMEM((_CHUNK,), jnp.float32),
            pltpu.VMEM((_TABLE,), jnp.float32),
            pltpu.VMEM((_CHUNK,), jnp.float32),
            pltpu.SemaphoreType.DMA,
            pltpu.SemaphoreType.DMA,
        ],
        compiler_params=pltpu.CompilerParams(needs_layout_passes=False),
    )
    return run(t, gamma)
